# Initial kernel scaffold; baseline (speedup 1.0000x reference)
import jax
import jax.numpy as jnp
from jax.experimental import pallas as pl


def kernel(x, embedding_weight):
    raise NotImplementedError("write your pallas kernel here")


# speedup vs baseline: 3.8698x; 3.8698x over previous
import functools

import jax
import jax.numpy as jnp
from jax import lax
from jax.experimental import pallas as pl
from jax.experimental.pallas import tpu as pltpu
from jax.experimental.pallas import tpu_sc as plsc

NC = 2
NS = 16
NW = NC * NS

L = 16
D = 10
N_TOTAL = 16384 * 200
CHUNK = 2048
GROUPS = CHUNK // L
PER_W = N_TOTAL // NW
N_CHUNKS = PER_W // CHUNK

_GDN = lax.GatherDimensionNumbers(
    offset_dims=(), collapsed_slice_dims=(0,), start_index_map=(0,)
)


def _lane_gather(vec, lanes):
    return lax.gather(
        vec,
        lanes.reshape(L, 1),
        dimension_numbers=_GDN,
        slice_sizes=(1,),
        mode=lax.GatherScatterMode.PROMISE_IN_BOUNDS,
    )


def _sc_embed(x_flat, table):
    mesh = plsc.VectorSubcoreMesh(
        core_axis_name="c", subcore_axis_name="s", num_cores=NC, num_subcores=NS
    )

    @functools.partial(
        pl.kernel,
        mesh=mesh,
        out_type=jax.ShapeDtypeStruct((N_TOTAL * D,), jnp.float32),
        scratch_types=[
            pltpu.VMEM((D, D), jnp.float32),
            pltpu.VMEM((CHUNK,), jnp.int32),
            pltpu.VMEM((CHUNK * D,), jnp.float32),
        ],
        compiler_params=pltpu.CompilerParams(
            use_tc_tiling_on_sc=False, needs_layout_passes=False
        ),
    )
    def k(idx_hbm, table_hbm, out_hbm, table_v, idx_v, rows_v):
        wid = lax.axis_index("s") * NC + lax.axis_index("c")
        base_w = wid * PER_W
        pltpu.sync_copy(table_hbm, table_v)

        lane = lax.iota(jnp.int32, L)
        divs = [(lane + L * t) // D for t in range(D)]
        mods = [(lane + L * t) % D for t in range(D)]

        def chunk_body(i, carry):
            base = base_w + i * CHUNK
            pltpu.sync_copy(idx_hbm.at[pl.ds(base, CHUNK)], idx_v)

            def group_body(g, c2):
                for t in range(D):
                    rows = plsc.load_gather(idx_v, [g * L + divs[t]])
                    vals = plsc.load_gather(table_v, [rows, mods[t]])
                    rows_v[pl.ds(g * L * D + L * t, L)] = vals
                return c2

            lax.fori_loop(0, GROUPS, group_body, 0)
            pltpu.sync_copy(rows_v, out_hbm.at[pl.ds(base * D, CHUNK * D)])
            return carry

        lax.fori_loop(0, N_CHUNKS, chunk_body, 0)

    return k(x_flat, table)


def kernel(x, embedding_weight):
    x_flat = x.reshape(-1).astype(jnp.int32)
    out = _sc_embed(x_flat, embedding_weight)
    return out.reshape(16384, 200, 10)


# speedup vs baseline: 14.9309x; 3.8583x over previous
import functools

import jax
import jax.numpy as jnp
from jax import lax
from jax.experimental import pallas as pl
from jax.experimental.pallas import tpu as pltpu
from jax.experimental.pallas import tpu_sc as plsc

NC = 2
NS = 16
NW = NC * NS

L = 16
D = 10
NB = 16384
NL = 200
BSTRIP = NB // NW
N_LT = NL // 8
CVECS = BSTRIP // L


def _sc_embed(x_t, table):
    mesh = plsc.VectorSubcoreMesh(
        core_axis_name="c", subcore_axis_name="s", num_cores=NC, num_subcores=NS
    )

    @functools.partial(
        pl.kernel,
        mesh=mesh,
        out_type=jax.ShapeDtypeStruct((D, NL, NB), jnp.float32),
        scratch_types=[
            pltpu.VMEM((D, D), jnp.float32),
            pltpu.VMEM((8, BSTRIP), jnp.int32),
            pltpu.VMEM((D, 8, BSTRIP), jnp.float32),
        ],
        compiler_params=pltpu.CompilerParams(needs_layout_passes=False),
    )
    def k(x_hbm, table_hbm, out_hbm, table_v, idx_v, out_v):
        wid = lax.axis_index("s") * NC + lax.axis_index("c")
        b0 = wid * BSTRIP
        pltpu.sync_copy(table_hbm, table_v)
        jvecs = [jnp.full((L,), j, jnp.int32) for j in range(D)]

        def lt_body(lt, carry):
            pltpu.sync_copy(
                x_hbm.at[pl.ds(lt * 8, 8), pl.ds(b0, BSTRIP)], idx_v
            )

            def c_body(c, c2):
                for s in range(8):
                    idx_vec = idx_v[s, pl.ds(c * L, L)]
                    for j in range(D):
                        vals = plsc.load_gather(table_v, [idx_vec, jvecs[j]])
                        out_v[j, s, pl.ds(c * L, L)] = vals
                return c2

            lax.fori_loop(0, CVECS, c_body, 0)
            for j in range(D):
                pltpu.sync_copy(
                    out_v.at[j],
                    out_hbm.at[j, pl.ds(lt * 8, 8), pl.ds(b0, BSTRIP)],
                )
            return carry

        lax.fori_loop(0, N_LT, lt_body, 0)

    return k(x_t, table)


def kernel(x, embedding_weight):
    x_t = jnp.swapaxes(x, 0, 1).astype(jnp.int32)
    out_t = _sc_embed(x_t, embedding_weight)
    return jnp.transpose(out_t, (2, 1, 0))


# speedup vs baseline: 16.3908x; 1.0978x over previous
import functools

import jax
import jax.numpy as jnp
from jax import lax
from jax.experimental import pallas as pl
from jax.experimental.pallas import tpu as pltpu
from jax.experimental.pallas import tpu_sc as plsc

NC = 2
NS = 16
NW = NC * NS

L = 16
D = 10
NB = 16384
NL = 200
BSTRIP = NB // NW
N_LT = NL // 8
CVECS = BSTRIP // L


def _sc_embed(x_t, table):
    mesh = plsc.VectorSubcoreMesh(
        core_axis_name="c", subcore_axis_name="s", num_cores=NC, num_subcores=NS
    )

    @functools.partial(
        pl.kernel,
        mesh=mesh,
        out_type=jax.ShapeDtypeStruct((D, NL, NB), jnp.float32),
        scratch_types=[
            pltpu.VMEM((D, D), jnp.float32),
            pltpu.VMEM((2, 8, BSTRIP), jnp.int32),
            pltpu.VMEM((2, D, 8, BSTRIP), jnp.float32),
            pltpu.SemaphoreType.DMA,
            pltpu.SemaphoreType.DMA,
            pltpu.SemaphoreType.DMA,
            pltpu.SemaphoreType.DMA,
        ],
        compiler_params=pltpu.CompilerParams(needs_layout_passes=False),
    )
    def k(x_hbm, table_hbm, out_hbm, table_v, idx_v, out_v,
          sem_in0, sem_in1, sem_out0, sem_out1):
        wid = lax.axis_index("s") * NC + lax.axis_index("c")
        b0 = wid * BSTRIP
        sems_in = (sem_in0, sem_in1)
        sems_out = (sem_out0, sem_out1)
        pltpu.sync_copy(table_hbm, table_v)
        jvecs = [jnp.full((L,), j, jnp.int32) for j in range(D)]

        def in_desc(lt, b):
            return pltpu.make_async_copy(
                x_hbm.at[pl.ds(lt * 8, 8), pl.ds(b0, BSTRIP)],
                idx_v.at[b], sems_in[b],
            )

        def out_desc(lt, b):
            return pltpu.make_async_copy(
                out_v.at[b],
                out_hbm.at[:, pl.ds(lt * 8, 8), pl.ds(b0, BSTRIP)],
                sems_out[b],
            )

        def compute(b):
            def c_body(c, c2):
                for s in range(8):
                    idx_vec = idx_v[b, s, pl.ds(c * L, L)]
                    for j in range(D):
                        vals = plsc.load_gather(table_v, [idx_vec, jvecs[j]])
                        out_v[b, j, s, pl.ds(c * L, L)] = vals
                return c2
            lax.fori_loop(0, CVECS, c_body, 0)

        in_desc(0, 0).start()
        in_desc(1, 1).start()

        def body(kk, carry):
            for b in range(2):
                lt = 2 * kk + b
                in_desc(0, b).wait()

                @pl.when(kk > 0)
                def _():
                    out_desc(0, b).wait()

                compute(b)
                out_desc(lt, b).start()

                nxt = lt + 2
                if b == 0:
                    in_desc(nxt, b).start()
                else:
                    @pl.when(kk < 11)
                    def _():
                        in_desc(nxt, b).start()
            return carry

        lax.fori_loop(0, 12, body, 0)

        in_desc(0, 0).wait()
        out_desc(0, 0).wait()
        compute(0)
        out_desc(24, 0).start()
        out_desc(0, 0).wait()
        out_desc(0, 1).wait()

    return k(x_t, table)


def kernel(x, embedding_weight):
    x_t = jnp.swapaxes(x, 0, 1).astype(jnp.int32)
    out_t = _sc_embed(x_t, embedding_weight)
    return jnp.transpose(out_t, (2, 1, 0))


# speedup vs baseline: 97.1627x; 5.9279x over previous
import functools

import jax
import jax.numpy as jnp
from jax import lax
from jax.experimental import pallas as pl
from jax.experimental.pallas import tpu as pltpu
from jax.experimental.pallas import tpu_sc as plsc

NC = 2
NS = 16
NW = NC * NS

L = 16
D = 10
NB = 16384
NL = 200
BSTRIP = NB // NW
N_LT = NL // 8
CVECS = BSTRIP // L


def _sc_embed(x_t, table):
    mesh = plsc.VectorSubcoreMesh(
        core_axis_name="c", subcore_axis_name="s", num_cores=NC, num_subcores=NS
    )

    @functools.partial(
        pl.kernel,
        mesh=mesh,
        out_type=jax.ShapeDtypeStruct((D, NL, NB), jnp.float32),
        scratch_types=[
            pltpu.VMEM((D * D, L), jnp.float32),
            pltpu.VMEM((2, 8, BSTRIP), jnp.int32),
            pltpu.VMEM((2, D, 8, BSTRIP), jnp.float32),
            pltpu.SemaphoreType.DMA,
            pltpu.SemaphoreType.DMA,
            pltpu.SemaphoreType.DMA,
            pltpu.SemaphoreType.DMA,
        ],
        compiler_params=pltpu.CompilerParams(needs_layout_passes=False),
    )
    def k(x_hbm, table_hbm, out_hbm, table_v, idx_v, out_v,
          sem_in0, sem_in1, sem_out0, sem_out1):
        wid = lax.axis_index("s") * NC + lax.axis_index("c")
        b0 = wid * BSTRIP
        sems_in = (sem_in0, sem_in1)
        sems_out = (sem_out0, sem_out1)
        pltpu.sync_copy(table_hbm, table_v)
        lane = lax.iota(jnp.int32, L)

        def in_desc(lt, b):
            return pltpu.make_async_copy(
                x_hbm.at[pl.ds(lt * 8, 8), pl.ds(b0, BSTRIP)],
                idx_v.at[b], sems_in[b],
            )

        def out_desc(lt, b):
            return pltpu.make_async_copy(
                out_v.at[b],
                out_hbm.at[:, pl.ds(lt * 8, 8), pl.ds(b0, BSTRIP)],
                sems_out[b],
            )

        def compute(b):
            @plsc.parallel_loop(0, CVECS, unroll=2)
            def c_body(c):
                for s in range(8):
                    idx_vec = idx_v[b, s, pl.ds(c * L, L)]
                    e_base = idx_vec * D
                    for j in range(D):
                        vals = plsc.load_gather(table_v, [e_base + j, lane])
                        out_v[b, j, s, pl.ds(c * L, L)] = vals

        in_desc(0, 0).start()
        in_desc(1, 1).start()

        def body(kk, carry):
            for b in range(2):
                lt = 2 * kk + b
                in_desc(0, b).wait()

                @pl.when(kk > 0)
                def _():
                    out_desc(0, b).wait()

                compute(b)
                out_desc(lt, b).start()

                nxt = lt + 2
                if b == 0:
                    in_desc(nxt, b).start()
                else:
                    @pl.when(kk < 11)
                    def _():
                        in_desc(nxt, b).start()
            return carry

        lax.fori_loop(0, 12, body, 0)

        in_desc(0, 0).wait()
        out_desc(0, 0).wait()
        compute(0)
        out_desc(24, 0).start()
        out_desc(0, 0).wait()
        out_desc(0, 1).wait()

    return k(x_t, table)


def kernel(x, embedding_weight):
    x_t = jnp.swapaxes(x, 0, 1).astype(jnp.int32)
    w_rep = jnp.broadcast_to(
        embedding_weight.astype(jnp.float32).reshape(D * D, 1), (D * D, L)
    )
    out_t = _sc_embed(x_t, w_rep)
    return jnp.transpose(out_t, (2, 1, 0))
